# baseline (speedup 1.0000x reference)
import functools

import jax
import jax.numpy as jnp
from jax import lax
from jax.experimental import pallas as pl
from jax.experimental.pallas import tpu as pltpu
from jax.experimental.pallas import tpu_sc as plsc

_D = 128
_B = 4096
_L = 16

_info = plsc.get_sparse_core_info()
_NC = _info.num_cores
_NS = _info.num_subcores
_NW = _NC * _NS
_BPW = _B // _NW
_VPR = _D // _L

_mesh = plsc.VectorSubcoreMesh(core_axis_name="c", subcore_axis_name="s")


@functools.partial(
    pl.kernel,
    mesh=_mesh,
    out_type=jax.ShapeDtypeStruct((_NW, _L), jnp.float32),
    scratch_types=[
        pltpu.VMEM((_BPW,), jnp.int32),
        pltpu.VMEM((_BPW, _D), jnp.float32),
        pltpu.VMEM((_BPW, _D), jnp.float32),
        pltpu.VMEM((_L,), jnp.float32),
        pltpu.SemaphoreType.DMA,
        pltpu.SemaphoreType.DMA,
    ],
)
def _loss_parts(target_hbm, emb_hbm, centers_hbm, out_hbm,
                idx_v, ctr_v, emb_v, acc_v, sem_e, sem_g):
    wid = lax.axis_index("s") * _NC + lax.axis_index("c")
    base = wid * _BPW
    emb_cp = pltpu.async_copy(emb_hbm.at[pl.ds(base, _BPW)], emb_v, sem_e)
    pltpu.sync_copy(target_hbm.at[pl.ds(base, _BPW)], idx_v)
    gather = pltpu.async_copy(centers_hbm.at[idx_v], ctr_v, sem_g)
    emb_cp.wait()
    gather.wait()

    zero = jnp.zeros((_L,), jnp.float32)

    @plsc.parallel_loop(0, _BPW, carry=(zero,) * _VPR)
    def accs(r, accs):
        new = []
        for j in range(_VPR):
            d = emb_v[r, pl.ds(j * _L, _L)] - ctr_v[r, pl.ds(j * _L, _L)]
            new.append(accs[j] + d * d)
        return tuple(new)
    acc = accs[0]
    for j in range(1, _VPR):
        acc = acc + accs[j]
    acc_v[...] = 0.5 * acc
    pltpu.sync_copy(acc_v, out_hbm.at[wid])


@jax.jit
def _center_loss(target, vector_embedding, centers):
    parts = _loss_parts(target, vector_embedding, centers)
    return jnp.sum(parts)


def kernel(target, vector_embedding, centers):
    return _center_loss(target.astype(jnp.int32), vector_embedding, centers)


# speedup vs baseline: 1.0031x; 1.0031x over previous
import functools

import jax
import jax.numpy as jnp
from jax import lax
from jax.experimental import pallas as pl
from jax.experimental.pallas import tpu as pltpu
from jax.experimental.pallas import tpu_sc as plsc

_D = 128
_B = 4096
_L = 16

_info = plsc.get_sparse_core_info()
_NC = _info.num_cores
_NS = _info.num_subcores
_NW = _NC * _NS
_BPW = _B // _NW
_VPR = _D // _L

_mesh = plsc.VectorSubcoreMesh(core_axis_name="c", subcore_axis_name="s")


@functools.partial(
    pl.kernel,
    mesh=_mesh,
    out_type=jax.ShapeDtypeStruct((_NW, _L), jnp.float32),
    scratch_types=[
        pltpu.VMEM((_BPW,), jnp.int32),
        pltpu.VMEM((_BPW, _D), jnp.float32),
        pltpu.VMEM((_BPW, _D), jnp.float32),
        pltpu.VMEM((_L,), jnp.float32),
        pltpu.SemaphoreType.DMA,
        pltpu.SemaphoreType.DMA,
    ],
)
def _loss_parts(target_hbm, emb_hbm, centers_hbm, out_hbm,
                idx_v, ctr_v, emb_v, acc_v, sem_e, sem_g):
    wid = lax.axis_index("s") * _NC + lax.axis_index("c")
    base = wid * _BPW
    emb_cp = pltpu.async_copy(emb_hbm.at[pl.ds(base, _BPW)], emb_v, sem_e)
    pltpu.sync_copy(target_hbm.at[pl.ds(base, _BPW)], idx_v)
    gather = pltpu.async_copy(centers_hbm.at[idx_v], ctr_v, sem_g)
    emb_cp.wait()
    gather.wait()

    def row(r, accs):
        new = []
        for j in range(_VPR):
            d = emb_v[r, pl.ds(j * _L, _L)] - ctr_v[r, pl.ds(j * _L, _L)]
            new.append(accs[j] + d * d)
        return tuple(new)

    zero = jnp.zeros((_L,), jnp.float32)
    accs = lax.fori_loop(0, _BPW, row, (zero,) * _VPR)
    acc = accs[0]
    for j in range(1, _VPR):
        acc = acc + accs[j]
    acc_v[...] = 0.5 * acc
    pltpu.sync_copy(acc_v, out_hbm.at[wid])


@jax.jit
def _center_loss(target, vector_embedding, centers):
    parts = _loss_parts(target, vector_embedding, centers)
    return jnp.sum(parts)


def kernel(target, vector_embedding, centers):
    return _center_loss(target.astype(jnp.int32), vector_embedding, centers)
